# baseline (speedup 1.0000x reference)
import functools

import jax
import jax.numpy as jnp
from jax.experimental import pallas as pl
from jax.experimental.pallas import tpu as pltpu

D = 64
QB = 1024
BLK = 2048
W = 128
CH = 512
TEMP = 0.1
PAD_VAL = 1.0e6


def _insert(a1, a2, a3, v):
    lo1 = jnp.minimum(a1, v)
    hi1 = jnp.maximum(a1, v)
    lo2 = jnp.minimum(a2, hi1)
    hi2 = jnp.maximum(a2, hi1)
    lo3 = jnp.minimum(a3, hi2)
    return lo1, lo2, lo3


def _knn_kernel(q_ref, m_ref, out_ref, t1_ref, t2_ref, t3_ref,
                u1_ref, u2_ref, u3_ref, u4_ref, *, nb):
    jb = pl.program_id(0)

    @pl.when(jb == 0)
    def _init():
        t1_ref[...] = jnp.full((QB, W), jnp.inf, jnp.float32)
        t2_ref[...] = jnp.full((QB, W), jnp.inf, jnp.float32)
        t3_ref[...] = jnp.full((QB, W), jnp.inf, jnp.float32)
        u1_ref[...] = jnp.full((QB, BLK // 2), jnp.inf, jnp.float32)
        u2_ref[...] = jnp.full((QB, BLK // 4), jnp.inf, jnp.float32)
        u3_ref[...] = jnp.full((QB, BLK // 8), jnp.inf, jnp.float32)
        u4_ref[...] = jnp.full((QB, BLK // 16), jnp.inf, jnp.float32)

    q = q_ref[...]
    q_aug = jnp.concatenate(
        [-2.0 * q, jnp.ones((QB, 1), jnp.float32)], axis=1)

    m = m_ref[...]
    m2 = jnp.sum(m * m, axis=1, keepdims=True)
    m_aug = jnp.concatenate([m, m2], axis=1)

    HW = CH // 2
    up_refs = (u2_ref, u3_ref)
    pend = [None] * 4
    for g in range(BLK // CH):
        mg_aug = m_aug[g * CH:(g + 1) * CH, :]
        sg = jax.lax.dot_general(
            q_aug, mg_aug, (((1,), (1,)), ((), ())),
            preferred_element_type=jnp.float32)
        a, b = sg[:, :HW], sg[:, HW:]
        sl = slice(g * HW, (g + 1) * HW)
        u1_ref[:, sl] = jnp.minimum(u1_ref[:, sl], jnp.maximum(a, b))
        lo = jnp.minimum(a, b)
        level, gg = 1, g
        while gg % 2 == 1:
            prev = pend[level]
            pend[level] = None
            slot = slice((gg // 2) * HW, (gg // 2 + 1) * HW)
            u_ref = up_refs[level - 1]
            u_ref[:, slot] = jnp.minimum(
                u_ref[:, slot], jnp.maximum(prev, lo))
            lo = jnp.minimum(prev, lo)
            level += 1
            gg //= 2
        pend[level] = lo

    root2 = pend[3]
    ra, rb = root2[:, :W], root2[:, W:]
    u4_ref[...] = jnp.minimum(u4_ref[...], jnp.maximum(ra, rb))
    root = jnp.minimum(ra, rb)
    b1, b2, b3 = _insert(t1_ref[...], t2_ref[...], t3_ref[...], root)
    t1_ref[...] = b1
    t2_ref[...] = b2
    t3_ref[...] = b3

    @pl.when(jb == nb - 1)
    def _finish():
        bb1, bb2, bb3 = b1, b2, b3
        for u_ref in (u4_ref, u3_ref, u2_ref, u1_ref):
            u = u_ref[...]
            for g in range(u.shape[1] // W):
                bb1, bb2, bb3 = _insert(
                    bb1, bb2, bb3, u[:, g * W:(g + 1) * W])
        cand = jnp.concatenate([bb1, bb2, bb3], axis=1)
        col = jax.lax.broadcasted_iota(jnp.int32, (QB, 3 * W), 1)
        c = cand
        vals = []
        for _ in range(3):
            idx = jnp.argmin(c, axis=1)[:, None]
            vals.append(jnp.min(c, axis=1, keepdims=True))
            c = jnp.where(col == idx, jnp.inf, c)
        q2 = jnp.sum(q * q, axis=1, keepdims=True)
        d = [jnp.sqrt(jnp.maximum(v + q2, 1e-12)) for v in vals]
        x = [-di / TEMP for di in d]
        xm = jnp.maximum(jnp.maximum(x[0], x[1]), x[2])
        e = [jnp.exp(xi - xm) for xi in x]
        z = e[0] + e[1] + e[2]
        out_ref[...] = (e[0] * d[0] + e[1] * d[1] + e[2] * d[2]) / z


@jax.jit
def _run(query, memory_bank):
    qrows = query.shape[0]
    mrows = memory_bank.shape[0]
    nb = pl.cdiv(mrows, BLK)
    mpad = nb * BLK
    if mpad != mrows:
        memory_bank = jnp.pad(
            memory_bank, ((0, mpad - mrows), (0, 0)), constant_values=PAD_VAL)
    out = pl.pallas_call(
        functools.partial(_knn_kernel, nb=nb),
        grid=(nb,),
        in_specs=[
            pl.BlockSpec((QB, D), lambda j: (0, 0)),
            pl.BlockSpec((BLK, D), lambda j: (j, 0)),
        ],
        out_specs=pl.BlockSpec((QB, 1), lambda j: (0, 0)),
        out_shape=jax.ShapeDtypeStruct((qrows, 1), jnp.float32),
        scratch_shapes=[
            pltpu.VMEM((QB, W), jnp.float32),
            pltpu.VMEM((QB, W), jnp.float32),
            pltpu.VMEM((QB, W), jnp.float32),
            pltpu.VMEM((QB, BLK // 2), jnp.float32),
            pltpu.VMEM((QB, BLK // 4), jnp.float32),
            pltpu.VMEM((QB, BLK // 8), jnp.float32),
            pltpu.VMEM((QB, BLK // 16), jnp.float32),
        ],
    )(query, memory_bank)
    return out[:, 0]


def kernel(query, memory_bank, k):
    del k
    return _run(query, memory_bank)


# speedup vs baseline: 1.0506x; 1.0506x over previous
import functools

import jax
import jax.numpy as jnp
from jax.experimental import pallas as pl
from jax.experimental.pallas import tpu as pltpu

D = 64
QB = 1024
BLK = 4096
W = 128
CH = 256
NLVL = 5
TEMP = 0.1
PAD_VAL = 1.0e6


def _insert(a1, a2, a3, v):
    lo1 = jnp.minimum(a1, v)
    hi1 = jnp.maximum(a1, v)
    lo2 = jnp.minimum(a2, hi1)
    hi2 = jnp.maximum(a2, hi1)
    lo3 = jnp.minimum(a3, hi2)
    return lo1, lo2, lo3


def _knn_kernel(q_ref, m_ref, out_ref, t1_ref, t2_ref, t3_ref, *u_refs,
                nb):
    jb = pl.program_id(0)

    @pl.when(jb == 0)
    def _init():
        t1_ref[...] = jnp.full((QB, W), jnp.inf, jnp.float32)
        t2_ref[...] = jnp.full((QB, W), jnp.inf, jnp.float32)
        t3_ref[...] = jnp.full((QB, W), jnp.inf, jnp.float32)
        for lvl, u_ref in enumerate(u_refs):
            u_ref[...] = jnp.full(
                (QB, BLK >> (lvl + 1)), jnp.inf, jnp.float32)

    q = q_ref[...]
    q_aug = jnp.concatenate(
        [-2.0 * q, jnp.ones((QB, 1), jnp.float32)], axis=1)

    pend = [None] * (NLVL + 1)
    for g in range(BLK // CH):
        mg = m_ref[pl.ds(g * CH, CH), :]
        m2g = jnp.sum(mg * mg, axis=1, keepdims=True)
        mg_aug = jnp.concatenate([mg, m2g], axis=1)
        sg = jax.lax.dot_general(
            q_aug, mg_aug, (((1,), (1,)), ((), ())),
            preferred_element_type=jnp.float32)
        a, b = sg[:, :W], sg[:, W:]
        sl = slice(g * W, (g + 1) * W)
        u1 = u_refs[0]
        u1[:, sl] = jnp.minimum(u1[:, sl], jnp.maximum(a, b))
        lo = jnp.minimum(a, b)
        level, gg = 1, g
        while gg % 2 == 1:
            prev = pend[level]
            pend[level] = None
            slot = slice((gg // 2) * W, (gg // 2 + 1) * W)
            u_ref = u_refs[level]
            u_ref[:, slot] = jnp.minimum(
                u_ref[:, slot], jnp.maximum(prev, lo))
            lo = jnp.minimum(prev, lo)
            level += 1
            gg //= 2
        pend[level] = lo

    root = pend[NLVL]
    b1, b2, b3 = _insert(t1_ref[...], t2_ref[...], t3_ref[...], root)
    t1_ref[...] = b1
    t2_ref[...] = b2
    t3_ref[...] = b3

    @pl.when(jb == nb - 1)
    def _finish():
        bb1, bb2, bb3 = b1, b2, b3
        for u_ref in u_refs[::-1]:
            u = u_ref[...]
            for g in range(u.shape[1] // W):
                bb1, bb2, bb3 = _insert(
                    bb1, bb2, bb3, u[:, g * W:(g + 1) * W])
        cand = jnp.concatenate([bb1, bb2, bb3], axis=1)
        col = jax.lax.broadcasted_iota(jnp.int32, (QB, 3 * W), 1)
        c = cand
        vals = []
        for _ in range(3):
            idx = jnp.argmin(c, axis=1)[:, None]
            vals.append(jnp.min(c, axis=1, keepdims=True))
            c = jnp.where(col == idx, jnp.inf, c)
        q2 = jnp.sum(q * q, axis=1, keepdims=True)
        d = [jnp.sqrt(jnp.maximum(v + q2, 1e-12)) for v in vals]
        x = [-di / TEMP for di in d]
        xm = jnp.maximum(jnp.maximum(x[0], x[1]), x[2])
        e = [jnp.exp(xi - xm) for xi in x]
        z = e[0] + e[1] + e[2]
        out_ref[...] = (e[0] * d[0] + e[1] * d[1] + e[2] * d[2]) / z


@jax.jit
def _run(query, memory_bank):
    qrows = query.shape[0]
    mrows = memory_bank.shape[0]
    nb = pl.cdiv(mrows, BLK)
    mpad = nb * BLK
    if mpad != mrows:
        memory_bank = jnp.pad(
            memory_bank, ((0, mpad - mrows), (0, 0)), constant_values=PAD_VAL)
    out = pl.pallas_call(
        functools.partial(_knn_kernel, nb=nb),
        grid=(nb,),
        in_specs=[
            pl.BlockSpec((QB, D), lambda j: (0, 0)),
            pl.BlockSpec((BLK, D), lambda j: (j, 0)),
        ],
        out_specs=pl.BlockSpec((QB, 1), lambda j: (0, 0)),
        out_shape=jax.ShapeDtypeStruct((qrows, 1), jnp.float32),
        scratch_shapes=[pltpu.VMEM((QB, W), jnp.float32)] * 3 + [
            pltpu.VMEM((QB, BLK >> (lvl + 1)), jnp.float32)
            for lvl in range(NLVL)
        ],
    )(query, memory_bank)
    return out[:, 0]


def kernel(query, memory_bank, k):
    del k
    return _run(query, memory_bank)
